# baseline (speedup 1.0000x reference)
import functools

import jax
import jax.numpy as jnp
from jax import lax
from jax.experimental import pallas as pl
from jax.experimental.pallas import tpu as pltpu
from jax.experimental.pallas import tpu_sc as plsc

B = 4096
L = 200
F = 64
VOCAB = 1000000
EPS = 1e-5

NC = 2
NS = 16
NW = NC * NS
BPW = B // NW
IPW = BPW * L

_mesh = plsc.VectorSubcoreMesh(core_axis_name="c", subcore_axis_name="s")


@functools.partial(
    pl.kernel,
    mesh=_mesh,
    compiler_params=pltpu.CompilerParams(use_tc_tiling_on_sc=True),
    out_type=jax.ShapeDtypeStruct((B * F,), jnp.float32),
    scratch_types=[
        pltpu.VMEM((IPW,), jnp.int32),
        pltpu.VMEM((L, 2 * F), jnp.float32),
        pltpu.VMEM((L, 2 * F), jnp.float32),
        pltpu.VMEM((BPW * F,), jnp.float32),
        pltpu.SMEM((L,), jnp.int32),
        pltpu.SMEM((L,), jnp.int32),
        pltpu.SemaphoreType.DMA,
        pltpu.SemaphoreType.DMA,
        pltpu.SemaphoreType.DMA,
    ],
)
def _pooled_sum_sc(x_hbm, table_hbm, out_hbm, idx_v, buf0, buf1, out_v,
                   par0, par1, semi, sem0, sem1):
    wid = lax.axis_index("s") * NC + lax.axis_index("c")
    pltpu.async_copy(x_hbm.at[pl.ds(wid * IPW, IPW)], idx_v, semi).wait()

    zero = jnp.zeros((16,), jnp.float32)

    def enqueue(r, buf, par, sem):
        base = r * L

        def one(idx, slot):
            q = lax.shift_right_logical(idx, 1)
            pltpu.async_copy(table_hbm.at[pl.ds(q, 1)],
                             buf.at[pl.ds(slot, 1)], sem)
            par[slot] = (idx & 1) * F

        def ek(k, _):
            vec = idx_v[pl.ds(base + k * 16, 16)]
            for t in range(16):
                one(vec[t], k * 16 + t)
            return 0
        lax.fori_loop(0, (L // 16), ek, 0)
        vec = idx_v[pl.ds(base + L - 16, 16)]
        for t in range(16 - (L % 16), 16):
            one(vec[t], L - 16 + t)

    def drain(buf, sem):
        pltpu.make_async_copy(table_hbm.at[pl.ds(0, L)], buf, sem).wait()

    def accumulate(buf, par, r):
        def jbody(j, carry):
            a0, a1, a2, a3 = carry
            row = buf.at[j]
            base = par[j]
            return (a0 + row[pl.ds(base, 16)],
                    a1 + row[pl.ds(base + 16, 16)],
                    a2 + row[pl.ds(base + 32, 16)],
                    a3 + row[pl.ds(base + 48, 16)])
        a0, a1, a2, a3 = lax.fori_loop(0, L, jbody, (zero, zero, zero, zero))
        out_v[pl.ds(r * F, 16)] = a0
        out_v[pl.ds(r * F + 16, 16)] = a1
        out_v[pl.ds(r * F + 32, 16)] = a2
        out_v[pl.ds(r * F + 48, 16)] = a3

    enqueue(0, buf0, par0, sem0)

    def body(i, _):
        r0 = 2 * i
        enqueue(r0 + 1, buf1, par1, sem1)
        drain(buf0, sem0)
        accumulate(buf0, par0, r0)

        @pl.when(r0 + 2 < BPW)
        def _():
            enqueue(r0 + 2, buf0, par0, sem0)
        drain(buf1, sem1)
        accumulate(buf1, par1, r0 + 1)
        return 0

    lax.fori_loop(0, BPW // 2, body, 0)
    pltpu.sync_copy(out_v, out_hbm.at[pl.ds(wid * BPW * F, BPW * F)])


def _tail_tc(ps_ref, w_ref, b_ref, bg_ref, bb_ref, lg_ref, lb_ref, o_ref):
    pooled = ps_ref[...] * (1.0 / L)
    h = lax.dot_general(pooled, w_ref[...], (((1,), (1,)), ((), ())),
                        preferred_element_type=jnp.float32) + b_ref[...]
    mu = jnp.mean(h, axis=0, keepdims=True)
    var = jnp.mean((h - mu) ** 2, axis=0, keepdims=True)
    h = (h - mu) * lax.rsqrt(var + EPS) * bg_ref[...] + bb_ref[...]
    lmu = jnp.mean(h, axis=-1, keepdims=True)
    lvar = jnp.mean((h - lmu) ** 2, axis=-1, keepdims=True)
    o_ref[...] = (h - lmu) * lax.rsqrt(lvar + EPS) * lg_ref[...] + lb_ref[...]


def kernel(x, table, W, b, bn_gamma, bn_beta, ln_gamma, ln_beta):
    xf = x.reshape(B * L).astype(jnp.int32)
    tablep = table.reshape(VOCAB // 2, 2 * F)
    pooled_sum = _pooled_sum_sc(xf, tablep).reshape(B, F)
    out = pl.pallas_call(
        _tail_tc,
        out_shape=jax.ShapeDtypeStruct((B, F), jnp.float32),
    )(pooled_sum, W, b.reshape(1, F), bn_gamma.reshape(1, F),
      bn_beta.reshape(1, F), ln_gamma.reshape(1, F), ln_beta.reshape(1, F))
    return out


# speedup vs baseline: 1.7987x; 1.7987x over previous
import functools

import jax
import jax.numpy as jnp
from jax import lax
from jax.experimental import pallas as pl
from jax.experimental.pallas import tpu as pltpu
from jax.experimental.pallas import tpu_sc as plsc

B = 4096
L = 200
F = 64
VOCAB = 1000000
EPS = 1e-5

NC = 2
NS = 16
NW = NC * NS
BPW = B // NW
IPW = BPW * L

_mesh = plsc.VectorSubcoreMesh(core_axis_name="c", subcore_axis_name="s")


@functools.partial(
    pl.kernel,
    mesh=_mesh,
    compiler_params=pltpu.CompilerParams(use_tc_tiling_on_sc=True),
    out_type=jax.ShapeDtypeStruct((B * F,), jnp.float32),
    scratch_types=[
        pltpu.VMEM((IPW,), jnp.int32),
        pltpu.VMEM((L, 2 * F), jnp.float32),
        pltpu.VMEM((L, 2 * F), jnp.float32),
        pltpu.VMEM((BPW * F,), jnp.float32),
        pltpu.SMEM((L,), jnp.int32),
        pltpu.SMEM((L,), jnp.int32),
        pltpu.SemaphoreType.DMA,
        pltpu.SemaphoreType.DMA,
        pltpu.SemaphoreType.DMA,
    ],
)
def _pooled_sum_sc(x_hbm, table_hbm, out_hbm, idx_v, buf0, buf1, out_v,
                   par0, par1, semi, sem0, sem1):
    wid = lax.axis_index("s") * NC + lax.axis_index("c")
    pltpu.async_copy(x_hbm.at[pl.ds(wid * IPW, IPW)], idx_v, semi).wait()

    zero = jnp.zeros((16,), jnp.float32)

    def enqueue(r, buf, par, sem):
        base = r * L

        def one(idx, slot):
            h = lax.shift_right_logical(idx, 15)
            r = idx & (VB - 1)
            q = h * HVB + (r & (HVB - 1))
            pltpu.async_copy(table_hbm.at[pl.ds(q, 1)],
                             buf.at[pl.ds(slot, 1)], sem)
            par[slot] = lax.shift_right_logical(r, 14) * F

        def ek(k, _):
            vec = idx_v[pl.ds(base + k * 16, 16)]
            for t in range(16):
                one(vec[t], k * 16 + t)
            return 0
        lax.fori_loop(0, (L // 16), ek, 0)
        vec = idx_v[pl.ds(base + L - 16, 16)]
        for t in range(16 - (L % 16), 16):
            one(vec[t], L - 16 + t)

    def drain(buf, sem):
        pltpu.make_async_copy(table_hbm.at[pl.ds(0, L)], buf, sem).wait()

    def accumulate(buf, par, r):
        def jbody(j, carry):
            a0, a1, a2, a3 = carry
            row = buf.at[j]
            base = par[j]
            return (a0 + row[pl.ds(base, 16)],
                    a1 + row[pl.ds(base + 16, 16)],
                    a2 + row[pl.ds(base + 32, 16)],
                    a3 + row[pl.ds(base + 48, 16)])
        a0, a1, a2, a3 = lax.fori_loop(0, L, jbody, (zero, zero, zero, zero))
        out_v[pl.ds(r * F, 16)] = a0
        out_v[pl.ds(r * F + 16, 16)] = a1
        out_v[pl.ds(r * F + 32, 16)] = a2
        out_v[pl.ds(r * F + 48, 16)] = a3

    enqueue(0, buf0, par0, sem0)

    def body(i, _):
        r0 = 2 * i
        enqueue(r0 + 1, buf1, par1, sem1)
        drain(buf0, sem0)
        accumulate(buf0, par0, r0)

        @pl.when(r0 + 2 < BPW)
        def _():
            enqueue(r0 + 2, buf0, par0, sem0)
        drain(buf1, sem1)
        accumulate(buf1, par1, r0 + 1)
        return 0

    lax.fori_loop(0, BPW // 2, body, 0)
    pltpu.sync_copy(out_v, out_hbm.at[pl.ds(wid * BPW * F, BPW * F)])


VB = 32768
HVB = VB // 2
NBLK = (VOCAB + VB - 1) // VB


def _pack_tc(tt_ref, o_ref):
    t = jnp.transpose(tt_ref[...])
    o_ref[...] = jnp.concatenate([t[:HVB, :], t[HVB:, :]], axis=1)


def _tail_tc(ps_ref, w_ref, b_ref, bg_ref, bb_ref, lg_ref, lb_ref, o_ref):
    pooled = ps_ref[...] * (1.0 / L)
    h = lax.dot_general(pooled, w_ref[...], (((1,), (1,)), ((), ())),
                        preferred_element_type=jnp.float32) + b_ref[...]
    mu = jnp.mean(h, axis=0, keepdims=True)
    var = jnp.mean((h - mu) ** 2, axis=0, keepdims=True)
    h = (h - mu) * lax.rsqrt(var + EPS) * bg_ref[...] + bb_ref[...]
    lmu = jnp.mean(h, axis=-1, keepdims=True)
    lvar = jnp.mean((h - lmu) ** 2, axis=-1, keepdims=True)
    o_ref[...] = (h - lmu) * lax.rsqrt(lvar + EPS) * lg_ref[...] + lb_ref[...]


def kernel(x, table, W, b, bn_gamma, bn_beta, ln_gamma, ln_beta):
    xf = x.reshape(B * L).astype(jnp.int32)
    tablep = pl.pallas_call(
        _pack_tc,
        grid=(NBLK,),
        in_specs=[pl.BlockSpec((F, VB), lambda j: (0, j))],
        out_specs=pl.BlockSpec((HVB, 2 * F), lambda j: (j, 0)),
        out_shape=jax.ShapeDtypeStruct((NBLK * HVB, 2 * F), jnp.float32),
    )(table.T)
    pooled_sum = _pooled_sum_sc(xf, tablep).reshape(B, F)
    out = pl.pallas_call(
        _tail_tc,
        out_shape=jax.ShapeDtypeStruct((B, F), jnp.float32),
    )(pooled_sum, W, b.reshape(1, F), bn_gamma.reshape(1, F),
      bn_beta.reshape(1, F), ln_gamma.reshape(1, F), ln_beta.reshape(1, F))
    return out
